# baseline (speedup 1.0000x reference)
import functools

import jax
import jax.numpy as jnp
from jax import lax
from jax.experimental import pallas as pl
from jax.experimental.pallas import tpu as pltpu
from jax.experimental.pallas import tpu_sc as plsc

N = 10000
E = 320000
D = 128
HP = 8
G = 16
PR = N // 16

NC = 2
NS = 16
NW = NC * NS
EPW = E // NW
RPT = N // NS


def _first_layer_body(x_ref, w_ref, b_ref, g_ref, be_ref, f_ref,
                      h_ref, z_ref):
    yp = jnp.dot(x_ref[:, 0, :], w_ref[0],
                 preferred_element_type=jnp.float32)
    for k in range(1, 16):
        yp = yp + jnp.dot(x_ref[:, k, :], w_ref[k],
                          preferred_element_type=jnp.float32)
    yp = yp + b_ref[...]
    s1 = jnp.sum(yp, axis=0, keepdims=True)
    mu = jnp.dot(s1, f_ref[...], preferred_element_type=jnp.float32) / N
    yc = yp - mu
    s2 = jnp.sum(yc * yc, axis=0, keepdims=True)
    var = jnp.dot(s2, f_ref[...], preferred_element_type=jnp.float32) / N
    r0 = lax.rsqrt(var + 1e-5)
    r = r0 * (1.5 - 0.5 * (var + 1e-5) * r0 * r0)
    hn = g_ref[...] * yc * r + be_ref[...]
    h_ref[...] = jnp.maximum(hn, 0.0)
    z_ref[...] = jnp.zeros((PR, 128), jnp.float32)


def _segment_sum_body(h_hbm, z_hbm, src_hbm, dst_hbm, out0_hbm, out1_hbm,
                      h_sh, agg_sh, src_v, dst_v, rows_v, sem):
    cid = lax.axis_index("c")
    sid = lax.axis_index("s")
    wid = sid * NC + cid
    rbase = sid * RPT

    pltpu.sync_copy(h_hbm.at[pl.ds(rbase, RPT)], h_sh.at[pl.ds(rbase, RPT)])
    pltpu.sync_copy(z_hbm.at[pl.ds(rbase, RPT)], agg_sh.at[pl.ds(rbase, RPT)])

    ebase = wid * EPW
    pltpu.sync_copy(src_hbm.at[pl.ds(ebase, EPW)], src_v)
    pltpu.sync_copy(dst_hbm.at[pl.ds(ebase, EPW)], dst_v)
    plsc.subcore_barrier()

    pltpu.async_copy(h_sh.at[src_v], rows_v, sem).wait()
    pltpu.sync_copy(rows_v, agg_sh.at[dst_v], add=True)

    plsc.subcore_barrier()

    @pl.when(cid == 0)
    def _():
        pltpu.sync_copy(agg_sh.at[pl.ds(rbase, RPT)], out0_hbm.at[pl.ds(rbase, RPT)])

    @pl.when(cid == 1)
    def _():
        pltpu.sync_copy(agg_sh.at[pl.ds(rbase, RPT)], out1_hbm.at[pl.ds(rbase, RPT)])


_segment_sum_sc = functools.partial(
    pl.kernel,
    mesh=plsc.VectorSubcoreMesh(
        core_axis_name="c", subcore_axis_name="s",
        num_cores=NC, num_subcores=NS),
    out_type=(
        jax.ShapeDtypeStruct((N, HP), jnp.float32),
        jax.ShapeDtypeStruct((N, HP), jnp.float32),
    ),
    compiler_params=pltpu.CompilerParams(use_tc_tiling_on_sc=False),
    scratch_types=[
        pltpu.VMEM_SHARED((N, HP), jnp.float32),
        pltpu.VMEM_SHARED((N, HP), jnp.float32),
        pltpu.VMEM((EPW,), jnp.int32),
        pltpu.VMEM((EPW,), jnp.int32),
        pltpu.VMEM((EPW, HP), jnp.float32),
        pltpu.SemaphoreType.DMA,
    ],
)(_segment_sum_body)


def _second_layer_body(h_ref, a0_ref, a1_ref, bp_ref,
                       wc_ref, bc_ref, gc_ref, bec_ref, f_ref, bsum_ref,
                       sel_ref, l0_ref, l0b_ref, l1_ref, l1b_ref,
                       wsi_ref, node_ref):
    h = h_ref[...]
    m = h + a0_ref[...] + a1_ref[...]
    y = jnp.dot(m, wc_ref[...], preferred_element_type=jnp.float32)
    y = y + bc_ref[...]
    s1 = jnp.sum(y, axis=0, keepdims=True)
    mu = jnp.dot(s1, f_ref[...], preferred_element_type=jnp.float32) / N
    yc = y - mu
    s2 = jnp.sum(yc * yc, axis=0, keepdims=True)
    var = jnp.dot(s2, f_ref[...], preferred_element_type=jnp.float32) / N
    r0 = lax.rsqrt(var + 1e-5)
    r = r0 * (1.5 - 0.5 * (var + 1e-5) * r0 * r0)
    h2 = gc_ref[...] * yc * r + bec_ref[...]
    h2 = jnp.maximum(h2, 0.0)
    np0 = jnp.dot(h * l0_ref[...], bsum_ref[...],
                  preferred_element_type=jnp.float32) + l0b_ref[...]
    np1 = jnp.dot(h2 * l1_ref[...], bsum_ref[...],
                  preferred_element_type=jnp.float32) + l1b_ref[...]
    node_ref[...] = jnp.dot(np0 + np1, sel_ref[...],
                            preferred_element_type=jnp.float32)
    bp = bp_ref[...]
    neg = jnp.float32(-jnp.inf)
    ws = []
    for g in range(G):
        msk = bp == g
        w0 = jnp.max(jnp.where(msk, np0, neg))
        w1 = jnp.max(jnp.where(msk, np1, neg))
        ws.append((w0 + w1)[None, None])
    wsi_ref[...] = jnp.concatenate(ws, axis=0)


def kernel(x, edge_index, batch, W1, b1, g1, be1, L0_W, L0_b,
           Wc, bc, gc, bec, L1_W, L1_b):
    f32 = jnp.float32
    def tile16(v):
        return jnp.tile(jnp.pad(v, (0, HP - v.shape[0])), 16)[None, :]

    W1p = jnp.pad(W1, ((0, 0), (0, HP - W1.shape[1])))
    kk = lax.broadcasted_iota(jnp.int32, (16, D, 128), 0)
    cc = lax.broadcasted_iota(jnp.int32, (16, D, 128), 2)
    w1_tiled = jnp.tile(W1p, (1, 16))[None, :, :]
    w16 = jnp.where(cc // HP == kk, w1_tiled, 0.0)
    b1t, g1t, be1t = tile16(b1), tile16(g1), tile16(be1)
    bct, gct, bect = tile16(bc), tile16(gc), tile16(bec)
    l0t = tile16(L0_W[:, 0])
    l1t = tile16(L1_W[:, 0])
    l0b = L0_b.reshape(1, 1).astype(f32)
    l1b = L1_b.reshape(1, 1).astype(f32)
    ii = lax.broadcasted_iota(jnp.int32, (128, 128), 0)
    jj = lax.broadcasted_iota(jnp.int32, (128, 128), 1)
    Wcp = jnp.pad(Wc, ((0, HP - Wc.shape[0]), (0, HP - Wc.shape[1])))
    wc_blk = jnp.where(ii // HP == jj // HP, jnp.tile(Wcp, (16, 16)), 0.0)
    fold = (ii % HP == jj % HP).astype(f32)
    bsum = (ii // HP == jj // HP).astype(f32)
    i2 = lax.broadcasted_iota(jnp.int32, (128, 16), 0)
    j2 = lax.broadcasted_iota(jnp.int32, (128, 16), 1)
    sel = (i2 == j2 * HP).astype(f32)
    bp = jnp.broadcast_to(batch.reshape(PR, 16, 1), (PR, 16, HP))
    bp = bp.reshape(PR, 128)

    h, z = pl.pallas_call(
        _first_layer_body,
        out_shape=(
            jax.ShapeDtypeStruct((PR, 128), f32),
            jax.ShapeDtypeStruct((PR, 128), f32),
        ),
    )(x.reshape(PR, 16, D), w16, b1t, g1t, be1t, fold)

    agg0, agg1 = _segment_sum_sc(h.reshape(N, HP), z.reshape(N, HP),
                                 edge_index[0], edge_index[1])
    a0p = agg0.reshape(PR, 128)
    a1p = agg1.reshape(PR, 128)

    wsi, node_sel = pl.pallas_call(
        _second_layer_body,
        out_shape=(
            jax.ShapeDtypeStruct((G, 1), f32),
            jax.ShapeDtypeStruct((PR, 16), f32),
        ),
    )(h, a0p, a1p, bp, wc_blk, bct, gct, bect, fold, bsum,
      sel, l0t, l0b, l1t, l1b)
    return (wsi, node_sel.reshape(N, 1))


# speedup vs baseline: 1.1741x; 1.1741x over previous
import functools

import jax
import jax.numpy as jnp
from jax import lax
from jax.experimental import pallas as pl
from jax.experimental.pallas import tpu as pltpu
from jax.experimental.pallas import tpu_sc as plsc

N = 10000
E = 320000
D = 128
HP = 8
G = 16
PR = N // 16

NC = 2
NS = 16
NW = NC * NS
EPW = E // NW
RPT = N // NS


NCHUNK = 5
CR = PR // NCHUNK


def _first_layer_body(x_ref, w_ref, b_ref, g_ref, be_ref, f_ref,
                      h_ref, z_ref, y_acc):
    i = pl.program_id(0)
    yp = jnp.dot(x_ref[:, 0, :], w_ref[0],
                 preferred_element_type=jnp.float32)
    for k in range(1, 16):
        yp = yp + jnp.dot(x_ref[:, k, :], w_ref[k],
                          preferred_element_type=jnp.float32)
    y_acc[pl.ds(i * CR, CR), :] = yp + b_ref[...]

    @pl.when(i == NCHUNK - 1)
    def _():
        yb = y_acc[...]
        s1 = jnp.sum(yb, axis=0, keepdims=True)
        mu = jnp.dot(s1, f_ref[...], preferred_element_type=jnp.float32) / N
        yc = yb - mu
        s2 = jnp.sum(yc * yc, axis=0, keepdims=True)
        var = jnp.dot(s2, f_ref[...], preferred_element_type=jnp.float32) / N
        r0 = lax.rsqrt(var + 1e-5)
        r = r0 * (1.5 - 0.5 * (var + 1e-5) * r0 * r0)
        hn = g_ref[...] * yc * r + be_ref[...]
        h_ref[...] = jnp.maximum(hn, 0.0)
        z_ref[...] = jnp.zeros((PR, 128), jnp.float32)


def _segment_sum_body(h_hbm, z_hbm, ei_hbm, out0_hbm, out1_hbm,
                      h_sh, agg_sh, src_v, dst_v, rows_v, sem):
    cid = lax.axis_index("c")
    sid = lax.axis_index("s")
    wid = sid * NC + cid
    rbase = sid * RPT

    pltpu.sync_copy(h_hbm.at[pl.ds(rbase, RPT)], h_sh.at[pl.ds(rbase, RPT)])
    pltpu.sync_copy(z_hbm.at[pl.ds(rbase, RPT)], agg_sh.at[pl.ds(rbase, RPT)])

    ebase = wid * EPW
    pltpu.sync_copy(ei_hbm.at[pl.ds(ebase, EPW)], src_v)
    pltpu.sync_copy(ei_hbm.at[pl.ds(E + ebase, EPW)], dst_v)
    plsc.subcore_barrier()

    pltpu.async_copy(h_sh.at[src_v], rows_v, sem).wait()
    pltpu.sync_copy(rows_v, agg_sh.at[dst_v], add=True)

    plsc.subcore_barrier()

    @pl.when(cid == 0)
    def _():
        pltpu.sync_copy(agg_sh.at[pl.ds(rbase, RPT)], out0_hbm.at[pl.ds(rbase, RPT)])

    @pl.when(cid == 1)
    def _():
        pltpu.sync_copy(agg_sh.at[pl.ds(rbase, RPT)], out1_hbm.at[pl.ds(rbase, RPT)])


_segment_sum_sc = functools.partial(
    pl.kernel,
    mesh=plsc.VectorSubcoreMesh(
        core_axis_name="c", subcore_axis_name="s",
        num_cores=NC, num_subcores=NS),
    out_type=(
        jax.ShapeDtypeStruct((N, HP), jnp.float32),
        jax.ShapeDtypeStruct((N, HP), jnp.float32),
    ),
    compiler_params=pltpu.CompilerParams(use_tc_tiling_on_sc=False),
    scratch_types=[
        pltpu.VMEM_SHARED((N, HP), jnp.float32),
        pltpu.VMEM_SHARED((N, HP), jnp.float32),
        pltpu.VMEM((EPW,), jnp.int32),
        pltpu.VMEM((EPW,), jnp.int32),
        pltpu.VMEM((EPW, HP), jnp.float32),
        pltpu.SemaphoreType.DMA,
    ],
)(_segment_sum_body)


def _second_layer_body(h_ref, a0_ref, a1_ref, bp_ref,
                       wc_ref, bc_ref, gc_ref, bec_ref, f_ref, bsum_ref,
                       sel_ref, l0_ref, l0b_ref, l1_ref, l1b_ref,
                       wsi_ref, node_ref):
    h = h_ref[...]
    m = h + a0_ref[...] + a1_ref[...]
    y = jnp.dot(m, wc_ref[...], preferred_element_type=jnp.float32)
    y = y + bc_ref[...]
    s1 = jnp.sum(y, axis=0, keepdims=True)
    mu = jnp.dot(s1, f_ref[...], preferred_element_type=jnp.float32) / N
    yc = y - mu
    s2 = jnp.sum(yc * yc, axis=0, keepdims=True)
    var = jnp.dot(s2, f_ref[...], preferred_element_type=jnp.float32) / N
    r0 = lax.rsqrt(var + 1e-5)
    r = r0 * (1.5 - 0.5 * (var + 1e-5) * r0 * r0)
    h2 = gc_ref[...] * yc * r + bec_ref[...]
    h2 = jnp.maximum(h2, 0.0)
    np0 = jnp.dot(h * l0_ref[...], bsum_ref[...],
                  preferred_element_type=jnp.float32) + l0b_ref[...]
    np1 = jnp.dot(h2 * l1_ref[...], bsum_ref[...],
                  preferred_element_type=jnp.float32) + l1b_ref[...]
    node_ref[...] = jnp.dot(np0 + np1, sel_ref[...],
                            preferred_element_type=jnp.float32)
    bp = bp_ref[...]
    neg = jnp.float32(-jnp.inf)
    ws = []
    for g in range(G):
        msk = bp == g
        w0 = jnp.max(jnp.where(msk, np0, neg))
        w1 = jnp.max(jnp.where(msk, np1, neg))
        ws.append((w0 + w1)[None, None])
    wsi_ref[...] = jnp.concatenate(ws, axis=0)


def kernel(x, edge_index, batch, W1, b1, g1, be1, L0_W, L0_b,
           Wc, bc, gc, bec, L1_W, L1_b):
    f32 = jnp.float32
    def tile16(v):
        return jnp.tile(jnp.pad(v, (0, HP - v.shape[0])), 16)[None, :]

    W1p = jnp.pad(W1, ((0, 0), (0, HP - W1.shape[1])))
    kk = lax.broadcasted_iota(jnp.int32, (16, D, 128), 0)
    cc = lax.broadcasted_iota(jnp.int32, (16, D, 128), 2)
    w1_tiled = jnp.tile(W1p, (1, 16))[None, :, :]
    w16 = jnp.where(cc // HP == kk, w1_tiled, 0.0)
    b1t, g1t, be1t = tile16(b1), tile16(g1), tile16(be1)
    bct, gct, bect = tile16(bc), tile16(gc), tile16(bec)
    l0t = tile16(L0_W[:, 0])
    l1t = tile16(L1_W[:, 0])
    l0b = L0_b.reshape(1, 1).astype(f32)
    l1b = L1_b.reshape(1, 1).astype(f32)
    ii = lax.broadcasted_iota(jnp.int32, (128, 128), 0)
    jj = lax.broadcasted_iota(jnp.int32, (128, 128), 1)
    Wcp = jnp.pad(Wc, ((0, HP - Wc.shape[0]), (0, HP - Wc.shape[1])))
    wc_blk = jnp.where(ii // HP == jj // HP, jnp.tile(Wcp, (16, 16)), 0.0)
    fold = (ii % HP == jj % HP).astype(f32)
    bsum = (ii // HP == jj // HP).astype(f32)
    i2 = lax.broadcasted_iota(jnp.int32, (128, 16), 0)
    j2 = lax.broadcasted_iota(jnp.int32, (128, 16), 1)
    sel = (i2 == j2 * HP).astype(f32)
    bp = jnp.broadcast_to(batch.reshape(PR, 16, 1), (PR, 16, HP))
    bp = bp.reshape(PR, 128)

    h, z = pl.pallas_call(
        _first_layer_body,
        grid=(NCHUNK,),
        in_specs=[
            pl.BlockSpec((CR, 16, D), lambda i: (i, 0, 0)),
            pl.BlockSpec((16, D, 128), lambda i: (0, 0, 0)),
            pl.BlockSpec((1, 128), lambda i: (0, 0)),
            pl.BlockSpec((1, 128), lambda i: (0, 0)),
            pl.BlockSpec((1, 128), lambda i: (0, 0)),
            pl.BlockSpec((128, 128), lambda i: (0, 0)),
        ],
        out_specs=(
            pl.BlockSpec((PR, 128), lambda i: (0, 0)),
            pl.BlockSpec((PR, 128), lambda i: (0, 0)),
        ),
        scratch_shapes=[pltpu.VMEM((PR, 128), f32)],
        out_shape=(
            jax.ShapeDtypeStruct((PR, 128), f32),
            jax.ShapeDtypeStruct((PR, 128), f32),
        ),
    )(x.reshape(PR, 16, D), w16, b1t, g1t, be1t, fold)

    ei = edge_index.reshape(2 * E)
    agg0, agg1 = _segment_sum_sc(h.reshape(N, HP), z.reshape(N, HP), ei)
    a0p = agg0.reshape(PR, 128)
    a1p = agg1.reshape(PR, 128)

    wsi, node_sel = pl.pallas_call(
        _second_layer_body,
        out_shape=(
            jax.ShapeDtypeStruct((G, 1), f32),
            jax.ShapeDtypeStruct((PR, 16), f32),
        ),
    )(h, a0p, a1p, bp, wc_blk, bct, gct, bect, fold, bsum,
      sel, l0t, l0b, l1t, l1b)
    return (wsi, node_sel.reshape(N, 1))


# speedup vs baseline: 1.2285x; 1.0463x over previous
import functools

import jax
import jax.numpy as jnp
from jax import lax
from jax.experimental import pallas as pl
from jax.experimental.pallas import tpu as pltpu
from jax.experimental.pallas import tpu_sc as plsc

N = 10000
E = 320000
D = 128
HP = 8
G = 16
PR = N // 16

NC = 2
NS = 16
NW = NC * NS
EPW = E // NW
RPT = N // NS


NCHUNK = 5
CR = PR // NCHUNK


def _first_layer_body(x_ref, w_ref, p_ref, f_ref, h_ref, z_ref, y_acc):
    i = pl.program_id(0)
    yp = jnp.dot(x_ref[:, 0, :], w_ref[0],
                 preferred_element_type=jnp.float32)
    for k in range(1, 16):
        yp = yp + jnp.dot(x_ref[:, k, :], w_ref[k],
                          preferred_element_type=jnp.float32)
    y_acc[pl.ds(i * CR, CR), :] = yp + p_ref[0:1, :]

    @pl.when(i == NCHUNK - 1)
    def _():
        yb = y_acc[...]
        s1 = jnp.sum(yb, axis=0, keepdims=True)
        mu = jnp.dot(s1, f_ref[...], preferred_element_type=jnp.float32) / N
        yc = yb - mu
        s2 = jnp.sum(yc * yc, axis=0, keepdims=True)
        var = jnp.dot(s2, f_ref[...], preferred_element_type=jnp.float32) / N
        r0 = lax.rsqrt(var + 1e-5)
        r = r0 * (1.5 - 0.5 * (var + 1e-5) * r0 * r0)
        hn = p_ref[1:2, :] * yc * r + p_ref[2:3, :]
        h_ref[...] = jnp.maximum(hn, 0.0)
        z_ref[...] = jnp.zeros((PR, 128), jnp.float32)


def _segment_sum_body(h_hbm, z_hbm, ei_hbm, out0_hbm, out1_hbm,
                      h_sh, agg_sh, src_v, dst_v, rows_v, sem, sem2):
    cid = lax.axis_index("c")
    sid = lax.axis_index("s")
    wid = sid * NC + cid
    rbase = sid * RPT

    pltpu.sync_copy(h_hbm.at[pl.ds(rbase, RPT)], h_sh.at[pl.ds(rbase, RPT)])
    pltpu.sync_copy(z_hbm.at[pl.ds(rbase, RPT)], agg_sh.at[pl.ds(rbase, RPT)])

    ebase = wid * EPW
    pltpu.sync_copy(ei_hbm.at[pl.ds(ebase, EPW)], src_v)
    pltpu.sync_copy(ei_hbm.at[pl.ds(E + ebase, EPW)], dst_v)
    plsc.subcore_barrier()

    HALF = EPW // 2
    pltpu.async_copy(h_sh.at[src_v.at[pl.ds(0, HALF)]],
                     rows_v.at[pl.ds(0, HALF)], sem).wait()
    pltpu.async_copy(h_sh.at[src_v.at[pl.ds(HALF, HALF)]],
                     rows_v.at[pl.ds(HALF, HALF)], sem2)
    pltpu.sync_copy(rows_v.at[pl.ds(0, HALF)],
                    agg_sh.at[dst_v.at[pl.ds(0, HALF)]], add=True)
    pltpu.make_async_copy(h_sh.at[src_v.at[pl.ds(HALF, HALF)]],
                          rows_v.at[pl.ds(HALF, HALF)], sem2).wait()
    pltpu.sync_copy(rows_v.at[pl.ds(HALF, HALF)],
                    agg_sh.at[dst_v.at[pl.ds(HALF, HALF)]], add=True)

    plsc.subcore_barrier()

    @pl.when(cid == 0)
    def _():
        pltpu.sync_copy(agg_sh.at[pl.ds(rbase, RPT)], out0_hbm.at[pl.ds(rbase, RPT)])

    @pl.when(cid == 1)
    def _():
        pltpu.sync_copy(agg_sh.at[pl.ds(rbase, RPT)], out1_hbm.at[pl.ds(rbase, RPT)])


_segment_sum_sc = functools.partial(
    pl.kernel,
    mesh=plsc.VectorSubcoreMesh(
        core_axis_name="c", subcore_axis_name="s",
        num_cores=NC, num_subcores=NS),
    out_type=(
        jax.ShapeDtypeStruct((N, HP), jnp.float32),
        jax.ShapeDtypeStruct((N, HP), jnp.float32),
    ),
    compiler_params=pltpu.CompilerParams(use_tc_tiling_on_sc=False),
    scratch_types=[
        pltpu.VMEM_SHARED((N, HP), jnp.float32),
        pltpu.VMEM_SHARED((N, HP), jnp.float32),
        pltpu.VMEM((EPW,), jnp.int32),
        pltpu.VMEM((EPW,), jnp.int32),
        pltpu.VMEM((EPW, HP), jnp.float32),
        pltpu.SemaphoreType.DMA,
        pltpu.SemaphoreType.DMA,
    ],
)(_segment_sum_body)


def _second_layer_body(h_ref, a0_ref, a1_ref, bp_ref,
                       wc_ref, p_ref, f_ref, bsum_ref,
                       sel_ref, l0b_ref, l1b_ref,
                       wsi_ref, node_ref):
    h = h_ref[...]
    m = h + a0_ref[...] + a1_ref[...]
    y = jnp.dot(m, wc_ref[...], preferred_element_type=jnp.float32)
    y = y + p_ref[3:4, :]
    s1 = jnp.sum(y, axis=0, keepdims=True)
    mu = jnp.dot(s1, f_ref[...], preferred_element_type=jnp.float32) / N
    yc = y - mu
    s2 = jnp.sum(yc * yc, axis=0, keepdims=True)
    var = jnp.dot(s2, f_ref[...], preferred_element_type=jnp.float32) / N
    r0 = lax.rsqrt(var + 1e-5)
    r = r0 * (1.5 - 0.5 * (var + 1e-5) * r0 * r0)
    h2 = p_ref[4:5, :] * yc * r + p_ref[5:6, :]
    h2 = jnp.maximum(h2, 0.0)
    np0 = jnp.dot(h * p_ref[6:7, :], bsum_ref[...],
                  preferred_element_type=jnp.float32) + l0b_ref[...]
    np1 = jnp.dot(h2 * p_ref[7:8, :], bsum_ref[...],
                  preferred_element_type=jnp.float32) + l1b_ref[...]
    node_ref[...] = jnp.dot(np0 + np1, sel_ref[...],
                            preferred_element_type=jnp.float32)
    bp = bp_ref[...]
    neg = jnp.float32(-jnp.inf)
    ws = []
    for g in range(G):
        msk = bp == g
        w0 = jnp.max(jnp.where(msk, np0, neg))
        w1 = jnp.max(jnp.where(msk, np1, neg))
        ws.append((w0 + w1)[None, None])
    wsi_ref[...] = jnp.concatenate(ws, axis=0)


def kernel(x, edge_index, batch, W1, b1, g1, be1, L0_W, L0_b,
           Wc, bc, gc, bec, L1_W, L1_b):
    f32 = jnp.float32
    W1p = jnp.pad(W1, ((0, 0), (0, HP - W1.shape[1])))
    kk = lax.broadcasted_iota(jnp.int32, (16, D, 128), 0)
    cc = lax.broadcasted_iota(jnp.int32, (16, D, 128), 2)
    w1_tiled = jnp.tile(W1p, (1, 16))[None, :, :]
    w16 = jnp.where(cc // HP == kk, w1_tiled, 0.0)
    p8 = jnp.stack([b1, g1, be1, bc, gc, bec, L0_W[:, 0], L1_W[:, 0]])
    params = jnp.tile(jnp.pad(p8, ((0, 0), (0, HP - p8.shape[1]))), (1, 16))
    l0b = L0_b.reshape(1, 1).astype(f32)
    l1b = L1_b.reshape(1, 1).astype(f32)
    ii = lax.broadcasted_iota(jnp.int32, (128, 128), 0)
    jj = lax.broadcasted_iota(jnp.int32, (128, 128), 1)
    Wcp = jnp.pad(Wc, ((0, HP - Wc.shape[0]), (0, HP - Wc.shape[1])))
    wc_blk = jnp.where(ii // HP == jj // HP, jnp.tile(Wcp, (16, 16)), 0.0)
    fold = (ii % HP == jj % HP).astype(f32)
    bsum = (ii // HP == jj // HP).astype(f32)
    i2 = lax.broadcasted_iota(jnp.int32, (128, 16), 0)
    j2 = lax.broadcasted_iota(jnp.int32, (128, 16), 1)
    sel = (i2 == j2 * HP).astype(f32)
    bp = jnp.broadcast_to(batch.reshape(PR, 16, 1), (PR, 16, HP))
    bp = bp.reshape(PR, 128)

    h, z = pl.pallas_call(
        _first_layer_body,
        grid=(NCHUNK,),
        in_specs=[
            pl.BlockSpec((CR, 16, D), lambda i: (i, 0, 0)),
            pl.BlockSpec((16, D, 128), lambda i: (0, 0, 0)),
            pl.BlockSpec((8, 128), lambda i: (0, 0)),
            pl.BlockSpec((128, 128), lambda i: (0, 0)),
        ],
        out_specs=(
            pl.BlockSpec((PR, 128), lambda i: (0, 0)),
            pl.BlockSpec((PR, 128), lambda i: (0, 0)),
        ),
        scratch_shapes=[pltpu.VMEM((PR, 128), f32)],
        out_shape=(
            jax.ShapeDtypeStruct((PR, 128), f32),
            jax.ShapeDtypeStruct((PR, 128), f32),
        ),
    )(x.reshape(PR, 16, D), w16, params, fold)

    ei = edge_index.reshape(2 * E)
    agg0, agg1 = _segment_sum_sc(h.reshape(N, HP), z.reshape(N, HP), ei)
    a0p = agg0.reshape(PR, 128)
    a1p = agg1.reshape(PR, 128)

    wsi, node_sel = pl.pallas_call(
        _second_layer_body,
        out_shape=(
            jax.ShapeDtypeStruct((G, 1), f32),
            jax.ShapeDtypeStruct((PR, 16), f32),
        ),
    )(h, a0p, a1p, bp, wc_blk, params, fold, bsum, sel, l0b, l1b)
    return (wsi, node_sel.reshape(N, 1))


# speedup vs baseline: 1.5472x; 1.2594x over previous
import functools

import jax
import jax.numpy as jnp
from jax import lax
from jax.experimental import pallas as pl
from jax.experimental.pallas import tpu as pltpu
from jax.experimental.pallas import tpu_sc as plsc

N = 10000
E = 320000
D = 128
HP = 8
G = 16
PR = N // 16

NC = 2
NS = 16
NW = NC * NS
EPW = E // NW
RPT = N // NS


NCHUNK = 5
CR = PR // NCHUNK


def _first_layer_body(x_ref, w_ref, p_ref, f_ref, h_ref, z_ref, y_acc):
    i = pl.program_id(0)
    yp = jnp.dot(x_ref[:, 0, :], w_ref[0],
                 preferred_element_type=jnp.float32)
    for k in range(1, 16):
        yp = yp + jnp.dot(x_ref[:, k, :], w_ref[k],
                          preferred_element_type=jnp.float32)
    y_acc[pl.ds(i * CR, CR), :] = yp + p_ref[0:1, :]

    @pl.when(i == NCHUNK - 1)
    def _():
        yb = y_acc[...]
        s1 = jnp.sum(yb, axis=0, keepdims=True)
        mu = jnp.dot(s1, f_ref[...], preferred_element_type=jnp.float32) / N
        yc = yb - mu
        s2 = jnp.sum(yc * yc, axis=0, keepdims=True)
        var = jnp.dot(s2, f_ref[...], preferred_element_type=jnp.float32) / N
        r0 = lax.rsqrt(var + 1e-5)
        r = r0 * (1.5 - 0.5 * (var + 1e-5) * r0 * r0)
        hn = p_ref[1:2, :] * yc * r + p_ref[2:3, :]
        h_ref[...] = jnp.maximum(hn, 0.0)
        z_ref[...] = jnp.zeros((PR, 128), jnp.float32)


def _segment_sum_body(h_hbm, z_hbm, ei_hbm, out0_hbm, out1_hbm,
                      h_sh, agg_sh, src_v, dst_v, rows_v,
                      sem, sem2, sem3, sem4):
    cid = lax.axis_index("c")
    sid = lax.axis_index("s")
    wid = sid * NC + cid
    rbase = sid * RPT

    ebase = wid * EPW
    c1 = pltpu.async_copy(h_hbm.at[pl.ds(rbase, RPT)],
                          h_sh.at[pl.ds(rbase, RPT)], sem)
    c2 = pltpu.async_copy(z_hbm.at[pl.ds(rbase, RPT)],
                          agg_sh.at[pl.ds(rbase, RPT)], sem2)
    c3 = pltpu.async_copy(ei_hbm.at[pl.ds(ebase, EPW)], src_v, sem3)
    c4 = pltpu.async_copy(ei_hbm.at[pl.ds(E + ebase, EPW)], dst_v, sem4)
    c1.wait(); c2.wait(); c3.wait(); c4.wait()
    plsc.subcore_barrier()

    HALF = EPW // 2
    pltpu.async_copy(h_sh.at[src_v.at[pl.ds(0, HALF)]],
                     rows_v.at[pl.ds(0, HALF)], sem).wait()
    pltpu.async_copy(h_sh.at[src_v.at[pl.ds(HALF, HALF)]],
                     rows_v.at[pl.ds(HALF, HALF)], sem2)
    pltpu.sync_copy(rows_v.at[pl.ds(0, HALF)],
                    agg_sh.at[dst_v.at[pl.ds(0, HALF)]], add=True)
    pltpu.make_async_copy(h_sh.at[src_v.at[pl.ds(HALF, HALF)]],
                          rows_v.at[pl.ds(HALF, HALF)], sem2).wait()
    pltpu.sync_copy(rows_v.at[pl.ds(HALF, HALF)],
                    agg_sh.at[dst_v.at[pl.ds(HALF, HALF)]], add=True)

    plsc.subcore_barrier()

    @pl.when(cid == 0)
    def _():
        pltpu.sync_copy(agg_sh.at[pl.ds(rbase, RPT)], out0_hbm.at[pl.ds(rbase, RPT)])

    @pl.when(cid == 1)
    def _():
        pltpu.sync_copy(agg_sh.at[pl.ds(rbase, RPT)], out1_hbm.at[pl.ds(rbase, RPT)])


_segment_sum_sc = functools.partial(
    pl.kernel,
    mesh=plsc.VectorSubcoreMesh(
        core_axis_name="c", subcore_axis_name="s",
        num_cores=NC, num_subcores=NS),
    out_type=(
        jax.ShapeDtypeStruct((N, HP), jnp.float32),
        jax.ShapeDtypeStruct((N, HP), jnp.float32),
    ),
    compiler_params=pltpu.CompilerParams(use_tc_tiling_on_sc=False),
    scratch_types=[
        pltpu.VMEM_SHARED((N, HP), jnp.float32),
        pltpu.VMEM_SHARED((N, HP), jnp.float32),
        pltpu.VMEM((EPW,), jnp.int32),
        pltpu.VMEM((EPW,), jnp.int32),
        pltpu.VMEM((EPW, HP), jnp.float32),
        pltpu.SemaphoreType.DMA,
        pltpu.SemaphoreType.DMA,
        pltpu.SemaphoreType.DMA,
        pltpu.SemaphoreType.DMA,
    ],
)(_segment_sum_body)


def _second_layer_body(h_ref, a0_ref, a1_ref, bp_ref,
                       wc_ref, p_ref, f_ref, bsum_ref,
                       sel_ref, l0b_ref, l1b_ref,
                       wsi_ref, node_ref):
    h = h_ref[...]
    m = h + a0_ref[...] + a1_ref[...]
    y = jnp.dot(m, wc_ref[...], preferred_element_type=jnp.float32)
    y = y + p_ref[3:4, :]
    s1 = jnp.sum(y, axis=0, keepdims=True)
    mu = jnp.dot(s1, f_ref[...], preferred_element_type=jnp.float32) / N
    yc = y - mu
    s2 = jnp.sum(yc * yc, axis=0, keepdims=True)
    var = jnp.dot(s2, f_ref[...], preferred_element_type=jnp.float32) / N
    r0 = lax.rsqrt(var + 1e-5)
    r = r0 * (1.5 - 0.5 * (var + 1e-5) * r0 * r0)
    h2 = p_ref[4:5, :] * yc * r + p_ref[5:6, :]
    h2 = jnp.maximum(h2, 0.0)
    np0 = jnp.dot(h * p_ref[6:7, :], bsum_ref[...],
                  preferred_element_type=jnp.float32) + l0b_ref[...]
    np1 = jnp.dot(h2 * p_ref[7:8, :], bsum_ref[...],
                  preferred_element_type=jnp.float32) + l1b_ref[...]
    node_ref[...] = jnp.dot(np0 + np1, sel_ref[...],
                            preferred_element_type=jnp.float32)
    bp = bp_ref[...]
    neg = jnp.float32(-jnp.inf)
    ws = []
    for g in range(G):
        msk = bp == g
        w0 = jnp.max(jnp.where(msk, np0, neg))
        w1 = jnp.max(jnp.where(msk, np1, neg))
        ws.append((w0 + w1)[None, None])
    wsi_ref[...] = jnp.concatenate(ws, axis=0)


def kernel(x, edge_index, batch, W1, b1, g1, be1, L0_W, L0_b,
           Wc, bc, gc, bec, L1_W, L1_b):
    f32 = jnp.float32
    W1p = jnp.pad(W1, ((0, 0), (0, HP - W1.shape[1])))
    kk = lax.broadcasted_iota(jnp.int32, (16, D, 128), 0)
    cc = lax.broadcasted_iota(jnp.int32, (16, D, 128), 2)
    w1_tiled = jnp.tile(W1p, (1, 16))[None, :, :]
    w16 = jnp.where(cc // HP == kk, w1_tiled, 0.0)
    p8 = jnp.stack([b1, g1, be1, bc, gc, bec, L0_W[:, 0], L1_W[:, 0]])
    params = jnp.tile(jnp.pad(p8, ((0, 0), (0, HP - p8.shape[1]))), (1, 16))
    l0b = L0_b.reshape(1, 1).astype(f32)
    l1b = L1_b.reshape(1, 1).astype(f32)
    ii = lax.broadcasted_iota(jnp.int32, (128, 128), 0)
    jj = lax.broadcasted_iota(jnp.int32, (128, 128), 1)
    Wcp = jnp.pad(Wc, ((0, HP - Wc.shape[0]), (0, HP - Wc.shape[1])))
    wc_blk = jnp.where(ii // HP == jj // HP, jnp.tile(Wcp, (16, 16)), 0.0)
    fold = (ii % HP == jj % HP).astype(f32)
    bsum = (ii // HP == jj // HP).astype(f32)
    i2 = lax.broadcasted_iota(jnp.int32, (128, 16), 0)
    j2 = lax.broadcasted_iota(jnp.int32, (128, 16), 1)
    sel = (i2 == j2 * HP).astype(f32)
    bp = jnp.broadcast_to(batch.reshape(PR, 16, 1), (PR, 16, HP))
    bp = bp.reshape(PR, 128)

    h, z = pl.pallas_call(
        _first_layer_body,
        grid=(NCHUNK,),
        in_specs=[
            pl.BlockSpec((CR, 16, D), lambda i: (i, 0, 0)),
            pl.BlockSpec((16, D, 128), lambda i: (0, 0, 0)),
            pl.BlockSpec((8, 128), lambda i: (0, 0)),
            pl.BlockSpec((128, 128), lambda i: (0, 0)),
        ],
        out_specs=(
            pl.BlockSpec((PR, 128), lambda i: (0, 0)),
            pl.BlockSpec((PR, 128), lambda i: (0, 0)),
        ),
        scratch_shapes=[pltpu.VMEM((PR, 128), f32)],
        out_shape=(
            jax.ShapeDtypeStruct((PR, 128), f32),
            jax.ShapeDtypeStruct((PR, 128), f32),
        ),
    )(x.reshape(PR, 16, D), w16, params, fold)

    ei = edge_index.reshape(2 * E)
    agg0, agg1 = _segment_sum_sc(h.reshape(N, HP), z.reshape(N, HP), ei)
    a0p = agg0.reshape(PR, 128)
    a1p = agg1.reshape(PR, 128)

    wsi, node_sel = pl.pallas_call(
        _second_layer_body,
        out_shape=(
            jax.ShapeDtypeStruct((G, 1), f32),
            jax.ShapeDtypeStruct((PR, 16), f32),
        ),
    )(h, a0p, a1p, bp, wc_blk, params, fold, bsum, sel, l0b, l1b)
    return (wsi, node_sel.reshape(N, 1))
